# baseline (speedup 1.0000x reference)
import functools

import jax
import jax.numpy as jnp
from jax import lax
from jax.experimental import pallas as pl
from jax.experimental.pallas import tpu as pltpu
from jax.experimental.pallas import tpu_sc as plsc

N = 10000
D = 128
NC = 2
NS = 16
DH = D // NC
CH = 128
KT = 160
EPT = KT * CH
EPAD = NS * EPT
NACC = 10240
RPT = NACC // NS
NBUF = 2
NPH = 2
HKT = KT // NPH


FPT = N // NS


def _sc_body(feat_hbm, src_hbm, dst_hbm, z2_hbm, z1_hbm, acc_out, deg_out,
             src_v, dst_v, rows_v, ones_v, sem0, sem1, dsem,
             feat_sh, acc_sh, deg_sh):
    cid = lax.axis_index("c")
    sid = lax.axis_index("s")
    r0 = sid * RPT
    f0 = sid * FPT
    sems = (sem0, sem1)

    pltpu.sync_copy(feat_hbm.at[pl.ds(cid * N + f0, FPT)],
                    feat_sh.at[pl.ds(f0, FPT)])
    for z in range(RPT // CH):
        pltpu.sync_copy(z2_hbm, acc_sh.at[pl.ds(r0 + z * CH, CH)])
    pltpu.sync_copy(z1_hbm, deg_sh.at[pl.ds(r0, RPT)])
    ones16 = jnp.full((16,), 1.0, dtype=jnp.float32)
    for g in range(CH // 16):
        ones_v[pl.ds(g * 16, 16)] = ones16
    plsc.subcore_barrier()

    for ph in range(NPH):
        pltpu.sync_copy(src_hbm.at[sid, pl.ds(ph * HKT, HKT)], src_v)
        pltpu.sync_copy(dst_hbm.at[sid, pl.ds(ph * HKT, HKT)], dst_v)

        for b in range(NBUF):
            pltpu.async_copy(feat_sh.at[src_v.at[b]], rows_v.at[b], sems[b])

        def outer(i, carry):
            for b in range(NBUF):
                j = i * NBUF + b
                buf = rows_v.at[b]
                pltpu.make_async_copy(feat_sh.at[pl.ds(0, CH)], buf,
                                      sems[b]).wait()

                @pl.when(cid == (b % 2))
                def _():
                    pltpu.async_copy(ones_v, deg_sh.at[dst_v.at[j]], dsem,
                                     add=True)

                pltpu.sync_copy(buf, acc_sh.at[dst_v.at[j]], add=True)

                @pl.when(j + NBUF < HKT)
                def _():
                    pltpu.async_copy(feat_sh.at[src_v.at[j + NBUF]], buf,
                                     sems[b])
            return carry

        lax.fori_loop(0, HKT // NBUF, outer, 0)

        pltpu.make_async_copy(src_hbm.at[0, pl.ds(0, HKT // 2)],
                              src_v.at[pl.ds(0, HKT // 2)], dsem).wait()

    plsc.subcore_barrier()
    pltpu.sync_copy(acc_sh.at[pl.ds(r0, RPT)],
                    acc_out.at[pl.ds(cid * NACC + r0, RPT)])
    pltpu.sync_copy(deg_sh.at[pl.ds(r0, RPT)],
                    deg_out.at[pl.ds(cid * NACC + r0, RPT)])


_sc_aggregate = functools.partial(
    pl.kernel,
    out_type=(
        jax.ShapeDtypeStruct((NC * NACC, DH), jnp.float32),
        jax.ShapeDtypeStruct((NC * NACC,), jnp.float32),
    ),
    mesh=plsc.VectorSubcoreMesh(
        core_axis_name="c", subcore_axis_name="s",
        num_cores=NC, num_subcores=NS),
    compiler_params=pltpu.CompilerParams(use_tc_tiling_on_sc=False),
    scratch_types=[
        pltpu.VMEM((HKT, CH), jnp.int32),
        pltpu.VMEM((HKT, CH), jnp.int32),
        pltpu.VMEM((NBUF, CH, DH), jnp.float32),
        pltpu.VMEM((CH,), jnp.float32),
        pltpu.SemaphoreType.DMA,
        pltpu.SemaphoreType.DMA,
        pltpu.SemaphoreType.DMA,
        pltpu.VMEM_SHARED((N, DH), jnp.float32),
        pltpu.VMEM_SHARED((NACC, DH), jnp.float32),
        pltpu.VMEM_SHARED((NACC,), jnp.float32),
    ],
)(_sc_body)


BN = 2000


def _tc_body(acc_ref, deg_ref, feat_ref, ws_ref, wn_ref, out_ref):
    s = jnp.concatenate((acc_ref[0], acc_ref[1]), axis=1)
    deg = deg_ref[...]
    degs = jnp.maximum(deg[:, 0:1] + deg[:, 1:2], 1.0)
    hn = s / degs
    dn = (((1,), (1,)), ((), ()))
    out_ref[...] = (
        lax.dot_general(feat_ref[...], ws_ref[...], dn,
                        preferred_element_type=jnp.float32)
        + lax.dot_general(hn, wn_ref[...], dn,
                          preferred_element_type=jnp.float32)
    )


_tc_combine = pl.pallas_call(
    _tc_body,
    grid=(N // BN,),
    in_specs=[
        pl.BlockSpec((NC, BN, DH), lambda i: (0, i, 0)),
        pl.BlockSpec((BN, NC), lambda i: (i, 0)),
        pl.BlockSpec((BN, D), lambda i: (i, 0)),
        pl.BlockSpec((D, D), lambda i: (0, 0)),
        pl.BlockSpec((D, D), lambda i: (0, 0)),
    ],
    out_specs=pl.BlockSpec((BN, D), lambda i: (i, 0)),
    out_shape=jax.ShapeDtypeStruct((N, D), jnp.float32),
)


def kernel(feat, edge_index, W_self, W_neigh, prev_layer_repeat, step, flag,
           reuse_embedding):
    E = edge_index.shape[1]
    pad = EPAD - E
    src = edge_index[0]
    dst = edge_index[1]
    pad_ar = jnp.arange(pad, dtype=jnp.int32)
    src_p = jnp.concatenate([src, pad_ar % N]).reshape(NS, KT, CH)
    dst_p = jnp.concatenate([dst, N + pad_ar % (NACC - N)]).reshape(NS, KT, CH)
    feat_halves = feat.reshape(N, NC, DH).transpose(1, 0, 2).reshape(NC * N, DH)

    zeros2d = jnp.zeros((CH, DH), jnp.float32)
    zeros1d = jnp.zeros((RPT,), jnp.float32)

    acc_flat, deg_flat = _sc_aggregate(feat_halves, src_p, dst_p,
                                       zeros2d, zeros1d)
    acc = acc_flat.reshape(NC, NACC, DH)
    deg = deg_flat.reshape(NC, NACC).T

    return _tc_combine(acc, deg, feat, W_self, W_neigh)


# speedup vs baseline: 1.3993x; 1.3993x over previous
import functools

import jax
import jax.numpy as jnp
from jax import lax
from jax.experimental import pallas as pl
from jax.experimental.pallas import tpu as pltpu
from jax.experimental.pallas import tpu_sc as plsc

N = 10000
D = 128
NC = 2
NS = 16
DH = D // NC
CH = 128
KT = 160
EPT = KT * CH
EPAD = NS * EPT
NACC = 10240
RPT = NACC // NS
NBUF = 4


def _sc_body(feat_hbm, src_hbm, dst_hbm, z2_hbm, z1_hbm, acc_out, deg_out,
             src_v, dst_v, rows_v, ones_v, sem0, sem1, sem2, sem3, dsem,
             acc_sh, deg_sh):
    cid = lax.axis_index("c")
    sid = lax.axis_index("s")
    w2 = cid * NS + sid
    r0 = sid * RPT
    sems = (sem0, sem1, sem2, sem3)

    pltpu.sync_copy(src_hbm.at[w2], src_v)
    pltpu.sync_copy(dst_hbm.at[sid], dst_v)
    pltpu.sync_copy(z2_hbm.at[pl.ds(r0, RPT)], acc_sh.at[pl.ds(r0, RPT)])
    pltpu.sync_copy(z1_hbm.at[pl.ds(r0, RPT)], deg_sh.at[pl.ds(r0, RPT)])
    ones16 = jnp.full((16,), 1.0, dtype=jnp.float32)
    for g in range(CH // 16):
        ones_v[pl.ds(g * 16, 16)] = ones16
    plsc.subcore_barrier()

    for b in range(NBUF):
        pltpu.async_copy(feat_hbm.at[src_v.at[b]], rows_v.at[b], sems[b])

    def outer(i, carry):
        for b in range(NBUF):
            j = i * NBUF + b
            buf = rows_v.at[b]
            pltpu.make_async_copy(feat_hbm.at[pl.ds(0, CH)], buf,
                                  sems[b]).wait()

            @pl.when(cid == (b % 2))
            def _():
                pltpu.async_copy(ones_v, deg_sh.at[dst_v.at[j]], dsem,
                                 add=True)

            pltpu.sync_copy(buf, acc_sh.at[dst_v.at[j]], add=True)

            @pl.when(j + NBUF < KT)
            def _():
                pltpu.async_copy(feat_hbm.at[src_v.at[j + NBUF]], buf,
                                 sems[b])
        return carry

    lax.fori_loop(0, KT // NBUF, outer, 0)

    pltpu.make_async_copy(src_hbm.at[0, pl.ds(0, KT // 2)],
                          src_v.at[pl.ds(0, KT // 2)], dsem).wait()

    plsc.subcore_barrier()
    pltpu.sync_copy(acc_sh.at[pl.ds(r0, RPT)],
                    acc_out.at[pl.ds(cid * NACC + r0, RPT)])
    pltpu.sync_copy(deg_sh.at[pl.ds(r0, RPT)],
                    deg_out.at[pl.ds(cid * NACC + r0, RPT)])


_sc_aggregate = functools.partial(
    pl.kernel,
    out_type=(
        jax.ShapeDtypeStruct((NC * NACC, DH), jnp.float32),
        jax.ShapeDtypeStruct((NC * NACC,), jnp.float32),
    ),
    mesh=plsc.VectorSubcoreMesh(
        core_axis_name="c", subcore_axis_name="s",
        num_cores=NC, num_subcores=NS),
    compiler_params=pltpu.CompilerParams(use_tc_tiling_on_sc=False),
    scratch_types=[
        pltpu.VMEM((KT, CH), jnp.int32),
        pltpu.VMEM((KT, CH), jnp.int32),
        pltpu.VMEM((NBUF, CH, DH), jnp.float32),
        pltpu.VMEM((CH,), jnp.float32),
        pltpu.SemaphoreType.DMA,
        pltpu.SemaphoreType.DMA,
        pltpu.SemaphoreType.DMA,
        pltpu.SemaphoreType.DMA,
        pltpu.SemaphoreType.DMA,
        pltpu.VMEM_SHARED((NACC, DH), jnp.float32),
        pltpu.VMEM_SHARED((NACC,), jnp.float32),
    ],
)(_sc_body)


BN = 2000


def _tc_body(acc_ref, deg_ref, feat_ref, ws_ref, wn_ref, out_ref):
    s = jnp.concatenate((acc_ref[0], acc_ref[1]), axis=1)
    deg = deg_ref[...]
    degs = jnp.maximum(deg[:, 0:1] + deg[:, 1:2], 1.0)
    hn = s / degs
    dn = (((1,), (1,)), ((), ()))
    out_ref[...] = (
        lax.dot_general(feat_ref[...], ws_ref[...], dn,
                        preferred_element_type=jnp.float32)
        + lax.dot_general(hn, wn_ref[...], dn,
                          preferred_element_type=jnp.float32)
    )


_tc_combine = pl.pallas_call(
    _tc_body,
    grid=(N // BN,),
    in_specs=[
        pl.BlockSpec((NC, BN, DH), lambda i: (0, i, 0)),
        pl.BlockSpec((BN, NC), lambda i: (i, 0)),
        pl.BlockSpec((BN, D), lambda i: (i, 0)),
        pl.BlockSpec((D, D), lambda i: (0, 0)),
        pl.BlockSpec((D, D), lambda i: (0, 0)),
    ],
    out_specs=pl.BlockSpec((BN, D), lambda i: (i, 0)),
    out_shape=jax.ShapeDtypeStruct((N, D), jnp.float32),
)


def kernel(feat, edge_index, W_self, W_neigh, prev_layer_repeat, step, flag,
           reuse_embedding):
    E = edge_index.shape[1]
    pad = EPAD - E
    src = edge_index[0]
    dst = edge_index[1]
    pad_ar = jnp.arange(pad, dtype=jnp.int32)
    src_p = jnp.concatenate([src, pad_ar % N]).reshape(NS, KT, CH)
    dst_p = jnp.concatenate([dst, N + pad_ar % (NACC - N)]).reshape(NS, KT, CH)
    src2 = jnp.concatenate([src_p, src_p + N]).astype(jnp.int32)
    feat_halves = feat.reshape(N, NC, DH).transpose(1, 0, 2).reshape(NC * N, DH)

    zeros2d = jnp.zeros((NACC, DH), jnp.float32)
    zeros1d = jnp.zeros((NACC,), jnp.float32)

    acc_flat, deg_flat = _sc_aggregate(feat_halves, src2, dst_p,
                                       zeros2d, zeros1d)
    acc = acc_flat.reshape(NC, NACC, DH)
    deg = deg_flat.reshape(NC, NACC).T

    return _tc_combine(acc, deg, feat, W_self, W_neigh)


# speedup vs baseline: 1.5907x; 1.1367x over previous
import functools

import jax
import jax.numpy as jnp
from jax import lax
from jax.experimental import pallas as pl
from jax.experimental.pallas import tpu as pltpu
from jax.experimental.pallas import tpu_sc as plsc

N = 10000
D = 128
NC = 2
NS = 16
DH = D // NC
CH = 128
KT = 160
EPT = KT * CH
EPAD = NS * EPT
NACC = 10240
RPT = NACC // NS
NBUF = 5


def _sc_body(feat_hbm, src_hbm, dst_hbm, z2_hbm, z1_hbm,
             acc0_out, acc1_out, deg0_out, deg1_out,
             src_v, dst_v, rows_v, ones_v, sem0, sem1, sem2, sem3,
             sem4, dsem, acc_sh, deg_sh):
    cid = lax.axis_index("c")
    sid = lax.axis_index("s")
    w2 = cid * NS + sid
    r0 = sid * RPT
    sems = (sem0, sem1, sem2, sem3, sem4)

    pltpu.sync_copy(src_hbm.at[w2], src_v)
    pltpu.sync_copy(dst_hbm.at[sid], dst_v)
    pltpu.sync_copy(z2_hbm.at[pl.ds(r0, RPT)], acc_sh.at[pl.ds(r0, RPT)])
    pltpu.sync_copy(z1_hbm.at[pl.ds(r0, RPT)], deg_sh.at[pl.ds(r0, RPT)])
    ones16 = jnp.full((16,), 1.0, dtype=jnp.float32)
    for g in range(CH // 16):
        ones_v[pl.ds(g * 16, 16)] = ones16
    plsc.subcore_barrier()

    for b in range(NBUF):
        pltpu.async_copy(feat_hbm.at[src_v.at[b]], rows_v.at[b], sems[b])

    def outer(i, carry):
        for b in range(NBUF):
            j = i * NBUF + b
            buf = rows_v.at[b]
            pltpu.make_async_copy(feat_hbm.at[pl.ds(0, CH)], buf,
                                  sems[b]).wait()

            @pl.when(cid == ((i * NBUF + b) % 2))
            def _():
                pltpu.async_copy(ones_v, deg_sh.at[dst_v.at[j]], dsem,
                                 add=True)

            pltpu.sync_copy(buf, acc_sh.at[dst_v.at[j]], add=True)

            @pl.when(j + NBUF < KT)
            def _():
                pltpu.async_copy(feat_hbm.at[src_v.at[j + NBUF]], buf,
                                 sems[b])
        return carry

    lax.fori_loop(0, KT // NBUF, outer, 0)

    pltpu.make_async_copy(src_hbm.at[0, pl.ds(0, KT // 2)],
                          src_v.at[pl.ds(0, KT // 2)], dsem).wait()

    plsc.subcore_barrier()

    @pl.when(cid == 0)
    def _():
        pltpu.sync_copy(acc_sh.at[pl.ds(r0, RPT)],
                        acc0_out.at[pl.ds(r0, RPT)])
        pltpu.sync_copy(deg_sh.at[pl.ds(r0, RPT)],
                        deg0_out.at[pl.ds(r0, RPT)])

    @pl.when(cid == 1)
    def _():
        pltpu.sync_copy(acc_sh.at[pl.ds(r0, RPT)],
                        acc1_out.at[pl.ds(r0, RPT)])
        pltpu.sync_copy(deg_sh.at[pl.ds(r0, RPT)],
                        deg1_out.at[pl.ds(r0, RPT)])


_sc_aggregate = functools.partial(
    pl.kernel,
    out_type=(
        jax.ShapeDtypeStruct((NACC, DH), jnp.float32),
        jax.ShapeDtypeStruct((NACC, DH), jnp.float32),
        jax.ShapeDtypeStruct((NACC,), jnp.float32),
        jax.ShapeDtypeStruct((NACC,), jnp.float32),
    ),
    mesh=plsc.VectorSubcoreMesh(
        core_axis_name="c", subcore_axis_name="s",
        num_cores=NC, num_subcores=NS),
    compiler_params=pltpu.CompilerParams(use_tc_tiling_on_sc=False),
    scratch_types=[
        pltpu.VMEM((KT, CH), jnp.int32),
        pltpu.VMEM((KT, CH), jnp.int32),
        pltpu.VMEM((NBUF, CH, DH), jnp.float32),
        pltpu.VMEM((CH,), jnp.float32),
        pltpu.SemaphoreType.DMA,
        pltpu.SemaphoreType.DMA,
        pltpu.SemaphoreType.DMA,
        pltpu.SemaphoreType.DMA,
        pltpu.SemaphoreType.DMA,
        pltpu.SemaphoreType.DMA,
        pltpu.VMEM_SHARED((NACC, DH), jnp.float32),
        pltpu.VMEM_SHARED((NACC,), jnp.float32),
    ],
)(_sc_body)


BN = 2000

_DN = (((1,), (1,)), ((), ()))


def _tc_self_body(feat_ref, ws_ref, out_ref):
    out_ref[...] = lax.dot_general(feat_ref[...], ws_ref[...], _DN,
                                   preferred_element_type=jnp.float32)


_tc_self = pl.pallas_call(
    _tc_self_body,
    grid=(N // BN,),
    in_specs=[
        pl.BlockSpec((BN, D), lambda i: (i, 0)),
        pl.BlockSpec((D, D), lambda i: (0, 0)),
    ],
    out_specs=pl.BlockSpec((BN, D), lambda i: (i, 0)),
    out_shape=jax.ShapeDtypeStruct((N, D), jnp.float32),
)


def _tc_body(acc0_ref, acc1_ref, deg_ref, self_ref, wn_ref, out_ref):
    s = jnp.concatenate((acc0_ref[...], acc1_ref[...]), axis=1)
    deg = deg_ref[...]
    degs = jnp.maximum(deg[:, 0:1] + deg[:, 1:2], 1.0)
    hn = s / degs
    out_ref[...] = self_ref[...] + lax.dot_general(
        hn, wn_ref[...], _DN, preferred_element_type=jnp.float32)


_tc_combine = pl.pallas_call(
    _tc_body,
    grid=(N // BN,),
    in_specs=[
        pl.BlockSpec((BN, DH), lambda i: (i, 0)),
        pl.BlockSpec((BN, DH), lambda i: (i, 0)),
        pl.BlockSpec((BN, NC), lambda i: (i, 0)),
        pl.BlockSpec((BN, D), lambda i: (i, 0)),
        pl.BlockSpec((D, D), lambda i: (0, 0)),
    ],
    out_specs=pl.BlockSpec((BN, D), lambda i: (i, 0)),
    out_shape=jax.ShapeDtypeStruct((N, D), jnp.float32),
)


def kernel(feat, edge_index, W_self, W_neigh, prev_layer_repeat, step, flag,
           reuse_embedding):
    E = edge_index.shape[1]
    pad = EPAD - E
    src = edge_index[0]
    dst = edge_index[1]
    pad_ar = jnp.arange(pad, dtype=jnp.int32)
    src_p = jnp.concatenate([src, pad_ar % N]).reshape(NS, KT, CH)
    dst_p = jnp.concatenate([dst, N + pad_ar % (NACC - N)]).reshape(NS, KT, CH)
    idx2 = 2 * src_p
    src2 = jnp.concatenate([idx2, idx2 + 1]).astype(jnp.int32)
    feat2 = feat.reshape(NC * N, DH)

    zeros2d = jnp.zeros((NACC, DH), jnp.float32)
    zeros1d = jnp.zeros((NACC,), jnp.float32)

    acc0, acc1, deg0, deg1 = _sc_aggregate(feat2, src2, dst_p,
                                           zeros2d, zeros1d)
    deg = jnp.stack([deg0, deg1], axis=1)
    self_part = _tc_self(feat, W_self)

    return _tc_combine(acc0, acc1, deg, self_part, W_neigh)
